# baseline (speedup 1.0000x reference)
import functools

import jax
import jax.numpy as jnp
from jax import lax
from jax.experimental import pallas as pl
from jax.experimental.pallas import tpu as pltpu
from jax.experimental.pallas import tpu_sc as plsc

E = 800000
N_NEI = 4
IN_C = 16
MID = 32
POOL = 32
CAT = 40

NW = 32
ROWS_W = E // NW
R = 256
NCHUNK = ROWS_W // R
R_TAIL = ROWS_W - NCHUNK * R

BM = 8000


def _proj_body(x_ref, w_ref, b_ref, s_ref, y_ref):
    x = x_ref[...]
    c = x.shape[1]
    s_ref[...] = (
        jnp.dot(x, w_ref[0:c, :], preferred_element_type=jnp.float32)
        + b_ref[...]
    )
    for j in range(N_NEI):
        y_ref[j] = jnp.dot(
            x, w_ref[(j + 1) * c:(j + 2) * c, :],
            preferred_element_type=jnp.float32,
        )


def _tc_projections(x, wstack, b):
    c = x.shape[1]
    s, ybig = pl.pallas_call(
        _proj_body,
        grid=(E // BM,),
        in_specs=[
            pl.BlockSpec((BM, c), lambda i: (i, 0)),
            pl.BlockSpec((5 * c, MID), lambda i: (0, 0)),
            pl.BlockSpec((1, MID), lambda i: (0, 0)),
        ],
        out_specs=(
            pl.BlockSpec((BM, MID), lambda i: (i, 0)),
            pl.BlockSpec((N_NEI, BM, MID), lambda i: (0, i, 0)),
        ),
        out_shape=(
            jax.ShapeDtypeStruct((E, MID), jnp.float32),
            jax.ShapeDtypeStruct((N_NEI, E, MID), jnp.float32),
        ),
    )(x, wstack, b.reshape(1, MID))
    return s, ybig.reshape(N_NEI * E, MID)


def _make_sc_combine(has_skip):
    mesh = plsc.VectorSubcoreMesh(core_axis_name="c", subcore_axis_name="s")

    def body(*refs):
        if has_skip:
            (s_hbm, ybig, he_hbm, skip_hbm, out_hbm,
             ilist, gb, sb, kb, sem) = refs
        else:
            (s_hbm, ybig, he_hbm, out_hbm,
             ilist, gb, sb, sem) = refs
            kb = None
        wid = lax.axis_index("s") * 2 + lax.axis_index("c")
        base = wid * ROWS_W

        iota = lax.iota(jnp.int32, 16)
        r4e = (iota & 3) * E

        def chunk(k, carry):
            off = base + jnp.minimum(k * R, ROWS_W - R)
            pltpu.sync_copy(he_hbm.at[pl.ds(off * 4, R * 4)], ilist)
            pltpu.sync_copy(s_hbm.at[pl.ds(off, R)], sb)
            if has_skip:
                pltpu.sync_copy(skip_hbm.at[pl.ds(off, R)], kb)

            def build(i, bcarry):
                sl = pl.ds(i * 16, 16)
                ilist[sl] = ilist[sl] + r4e
                return bcarry

            lax.fori_loop(0, R * 4 // 16, build, 0, unroll=8)

            cps = [
                pltpu.async_copy(
                    ybig.at[ilist.at[pl.ds(q * R, R)]],
                    gb.at[pl.ds(q * R, R)], sem)
                for q in range(N_NEI)
            ]
            for cp in cps:
                cp.wait()

            def row(r, rcarry):
                for c in (0, 16):
                    sl = pl.ds(c, 16)
                    v = (sb[r, sl] + gb[4 * r, sl] + gb[4 * r + 1, sl]
                         + gb[4 * r + 2, sl] + gb[4 * r + 3, sl])
                    v = jnp.maximum(v, 0.0)
                    if has_skip:
                        v = jnp.maximum(v + kb[r, sl], 0.0)
                    sb[r, sl] = v
                return rcarry

            lax.fori_loop(0, R, row, 0, unroll=2)
            pltpu.sync_copy(sb, out_hbm.at[pl.ds(off, R)])
            return carry

        nch = NCHUNK + (1 if R_TAIL else 0)
        lax.fori_loop(0, nch, chunk, 0)

    scratch = [
        pltpu.VMEM((N_NEI * R,), jnp.int32),
        pltpu.VMEM((N_NEI * R, MID), jnp.float32),
        pltpu.VMEM((R, MID), jnp.float32),
    ]
    if has_skip:
        scratch.append(pltpu.VMEM((R, MID), jnp.float32))
    scratch.append(pltpu.SemaphoreType.DMA)

    return functools.partial(
        pl.kernel,
        mesh=mesh,
        out_type=jax.ShapeDtypeStruct((E, MID), jnp.float32),
        scratch_types=scratch,
        compiler_params=pltpu.CompilerParams(use_tc_tiling_on_sc=False),
    )(body)


_sc_combine = _make_sc_combine(False)
_sc_combine_skip = _make_sc_combine(True)


def _pool_body(h_ref, wf_ref, bf_ref, o_ref):
    p = pl.program_id(0)
    m = jnp.mean(h_ref[...], axis=0).reshape(1, MID)
    part = jnp.dot(m, wf_ref[...], preferred_element_type=jnp.float32)

    @pl.when(p == 0)
    def _():
        o_ref[...] = bf_ref[...]

    o_ref[...] += part


def _pool_fc(h, wf, bf):
    seg = E // POOL
    out = pl.pallas_call(
        _pool_body,
        grid=(POOL,),
        in_specs=[
            pl.BlockSpec((seg, MID), lambda p: (p, 0)),
            pl.BlockSpec((MID, CAT), lambda p: (p, 0)),
            pl.BlockSpec((1, CAT), lambda p: (0, 0)),
        ],
        out_specs=pl.BlockSpec((1, CAT), lambda p: (0, 0)),
        out_shape=jax.ShapeDtypeStruct((1, CAT), jnp.float32),
    )(h, wf, bf.reshape(1, CAT))
    return out.reshape(CAT)


def _conv(x, he_flat, w, b, skip=None):
    s, ybig = _tc_projections(x, w, b)
    if skip is None:
        return _sc_combine(s, ybig, he_flat)
    return _sc_combine_skip(s, ybig, he_flat, skip)


def kernel(x, half_edges, W0, b0, W11, b11, W12, b12, W21, b21, W22, b22, Wf, bf):
    he_flat = half_edges.reshape(N_NEI * E)
    h = _conv(x, he_flat, W0, b0)
    for (wa, ba, wb, bb) in ((W11, b11, W12, b12), (W21, b21, W22, b22)):
        y = _conv(h, he_flat, wa, ba)
        h = _conv(y, he_flat, wb, bb, skip=h)
    return _pool_fc(h, Wf, bf)


# speedup vs baseline: 1.0469x; 1.0469x over previous
import functools

import jax
import jax.numpy as jnp
from jax import lax
from jax.experimental import pallas as pl
from jax.experimental.pallas import tpu as pltpu
from jax.experimental.pallas import tpu_sc as plsc

E = 800000
N_NEI = 4
IN_C = 16
MID = 32
POOL = 32
CAT = 40

NW = 32
ROWS_W = E // NW
R = 256
NCHUNK = ROWS_W // R
R_TAIL = ROWS_W - NCHUNK * R

BM = 8000


def _proj_body(x_ref, w_ref, b_ref, s_ref, y0_ref, y1_ref, y2_ref, y3_ref):
    x = x_ref[...]
    c = x.shape[1]
    s_ref[...] = (
        jnp.dot(x, w_ref[0:c, :], preferred_element_type=jnp.float32)
        + b_ref[...]
    )
    for j, y_ref in enumerate((y0_ref, y1_ref, y2_ref, y3_ref)):
        y_ref[...] = jnp.dot(
            x, w_ref[(j + 1) * c:(j + 2) * c, :],
            preferred_element_type=jnp.float32,
        )


def _tc_projections(x, wstack, b):
    c = x.shape[1]
    outs = pl.pallas_call(
        _proj_body,
        grid=(E // BM,),
        in_specs=[
            pl.BlockSpec((BM, c), lambda i: (i, 0)),
            pl.BlockSpec((5 * c, MID), lambda i: (0, 0)),
            pl.BlockSpec((1, MID), lambda i: (0, 0)),
        ],
        out_specs=tuple(pl.BlockSpec((BM, MID), lambda i: (i, 0))
                        for _ in range(5)),
        out_shape=tuple(jax.ShapeDtypeStruct((E, MID), jnp.float32)
                        for _ in range(5)),
    )(x, wstack, b.reshape(1, MID))
    return outs


def _make_sc_combine(has_skip):
    mesh = plsc.VectorSubcoreMesh(core_axis_name="c", subcore_axis_name="s")

    def body(*refs):
        if has_skip:
            (s_hbm, y0h, y1h, y2h, y3h, i0h, i1h, i2h, i3h, skip_hbm, out_hbm,
             i0, i1, i2, i3, g0, g1, g2, g3, sb, kb, isem, gsem) = refs
        else:
            (s_hbm, y0h, y1h, y2h, y3h, i0h, i1h, i2h, i3h, out_hbm,
             i0, i1, i2, i3, g0, g1, g2, g3, sb, isem, gsem) = refs
            kb = None
        wid = lax.axis_index("s") * 2 + lax.axis_index("c")
        base = wid * ROWS_W

        def chunk(k, carry):
            off = base + jnp.minimum(k * R, ROWS_W - R)
            sl_in = pl.ds(off, R)
            ics = [pltpu.async_copy(ih.at[sl_in], ib, isem)
                   for ih, ib in ((i0h, i0), (i1h, i1), (i2h, i2), (i3h, i3))]
            for ic in ics:
                ic.wait()
            gcs = [pltpu.async_copy(yh.at[ib], gb, gsem)
                   for yh, ib, gb in ((y0h, i0, g0), (y1h, i1, g1),
                                      (y2h, i2, g2), (y3h, i3, g3))]
            pltpu.sync_copy(s_hbm.at[sl_in], sb)
            if has_skip:
                pltpu.sync_copy(skip_hbm.at[sl_in], kb)
            for gc in gcs:
                gc.wait()

            def row(r, rcarry):
                for c in (0, 16):
                    sl = pl.ds(c, 16)
                    v = (sb[r, sl] + g0[r, sl] + g1[r, sl]
                         + g2[r, sl] + g3[r, sl])
                    v = jnp.maximum(v, 0.0)
                    if has_skip:
                        v = jnp.maximum(v + kb[r, sl], 0.0)
                    sb[r, sl] = v
                return rcarry

            lax.fori_loop(0, R, row, 0, unroll=2)
            pltpu.sync_copy(sb, out_hbm.at[sl_in])
            return carry

        nch = NCHUNK + (1 if R_TAIL else 0)
        lax.fori_loop(0, nch, chunk, 0)

    scratch = [pltpu.VMEM((R,), jnp.int32)] * 4
    scratch += [pltpu.VMEM((R, MID), jnp.float32)] * 5
    if has_skip:
        scratch.append(pltpu.VMEM((R, MID), jnp.float32))
    scratch += [pltpu.SemaphoreType.DMA, pltpu.SemaphoreType.DMA]

    return functools.partial(
        pl.kernel,
        mesh=mesh,
        out_type=jax.ShapeDtypeStruct((E, MID), jnp.float32),
        scratch_types=scratch,
        compiler_params=pltpu.CompilerParams(use_tc_tiling_on_sc=False),
    )(body)


_sc_combine = _make_sc_combine(False)
_sc_combine_skip = _make_sc_combine(True)


def _pool_body(h_ref, wf_ref, bf_ref, o_ref):
    p = pl.program_id(0)
    m = jnp.mean(h_ref[...], axis=0).reshape(1, MID)
    part = jnp.dot(m, wf_ref[...], preferred_element_type=jnp.float32)

    @pl.when(p == 0)
    def _():
        o_ref[...] = bf_ref[...]

    o_ref[...] += part


def _pool_fc(h, wf, bf):
    seg = E // POOL
    out = pl.pallas_call(
        _pool_body,
        grid=(POOL,),
        in_specs=[
            pl.BlockSpec((seg, MID), lambda p: (p, 0)),
            pl.BlockSpec((MID, CAT), lambda p: (p, 0)),
            pl.BlockSpec((1, CAT), lambda p: (0, 0)),
        ],
        out_specs=pl.BlockSpec((1, CAT), lambda p: (0, 0)),
        out_shape=jax.ShapeDtypeStruct((1, CAT), jnp.float32),
    )(h, wf, bf.reshape(1, CAT))
    return out.reshape(CAT)


def _conv(x, idx_cols, w, b, skip=None):
    s, y0, y1, y2, y3 = _tc_projections(x, w, b)
    if skip is None:
        return _sc_combine(s, y0, y1, y2, y3, *idx_cols)
    return _sc_combine_skip(s, y0, y1, y2, y3, *idx_cols, skip)


def kernel(x, half_edges, W0, b0, W11, b11, W12, b12, W21, b21, W22, b22, Wf, bf):
    idx_cols = [half_edges[:, j].reshape(E) for j in range(N_NEI)]
    h = _conv(x, idx_cols, W0, b0)
    for (wa, ba, wb, bb) in ((W11, b11, W12, b12), (W21, b21, W22, b22)):
        y = _conv(h, idx_cols, wa, ba)
        h = _conv(y, idx_cols, wb, bb, skip=h)
    return _pool_fc(h, Wf, bf)


# speedup vs baseline: 2.0231x; 1.9325x over previous
import functools

import jax
import jax.numpy as jnp
from jax import lax
from jax.experimental import pallas as pl
from jax.experimental.pallas import tpu as pltpu
from jax.experimental.pallas import tpu_sc as plsc

E = 800000
N_NEI = 4
IN_C = 16
MID = 32
POOL = 32
CAT = 40

NW = 32
ROWS_W = E // NW
R = 256
NCHUNK = ROWS_W // R
R_TAIL = ROWS_W - NCHUNK * R

BM = 8000
BMP = BM // 4
EP = E // 4


def _proj_body(x_ref, ws_ref, wj_ref, b_ref, s_ref, y_ref):
    x = x_ref[...]
    j = pl.program_id(1)

    @pl.when(j == 0)
    def _():
        s_ref[...] = (
            jnp.dot(x, ws_ref[...], preferred_element_type=jnp.float32)
            + b_ref[...]
        )

    y_ref[...] = jnp.dot(x, wj_ref[...], preferred_element_type=jnp.float32)


def _tc_projections(x_pk, wstack, b4):
    nb = E // BM
    s, ybig = pl.pallas_call(
        _proj_body,
        grid=(nb, N_NEI),
        in_specs=[
            pl.BlockSpec((BMP, 128), lambda i, j: (i, 0)),
            pl.BlockSpec((128, 128), lambda i, j: (0, 0)),
            pl.BlockSpec((128, 128), lambda i, j: (j + 1, 0)),
            pl.BlockSpec((1, 128), lambda i, j: (0, 0)),
        ],
        out_specs=(
            pl.BlockSpec((BMP, 128), lambda i, j: (i, 0)),
            pl.BlockSpec((BMP, 128), lambda i, j: (j * nb + i, 0)),
        ),
        out_shape=(
            jax.ShapeDtypeStruct((EP, 128), jnp.float32),
            jax.ShapeDtypeStruct((N_NEI * EP, 128), jnp.float32),
        ),
    )(x_pk, wstack, wstack, b4)
    return s.reshape(E, MID), ybig.reshape(N_NEI * E, MID)


def _make_sc_combine(has_skip):
    mesh = plsc.VectorSubcoreMesh(core_axis_name="c", subcore_axis_name="s")

    def body(*refs):
        if has_skip:
            (s_hbm, ybig, he_hbm, skip_hbm, out_hbm,
             ilist, gb, sb, kb, sem) = refs
        else:
            (s_hbm, ybig, he_hbm, out_hbm,
             ilist, gb, sb, sem) = refs
            kb = None
        wid = lax.axis_index("s") * 2 + lax.axis_index("c")
        base = wid * ROWS_W

        iota = lax.iota(jnp.int32, 16)
        r4e = (iota & 3) * E

        def chunk(k, carry):
            off = base + jnp.minimum(k * R, ROWS_W - R)
            pltpu.sync_copy(he_hbm.at[pl.ds(off * 4, R * 4)], ilist)
            pltpu.sync_copy(s_hbm.at[pl.ds(off, R)], sb)
            if has_skip:
                pltpu.sync_copy(skip_hbm.at[pl.ds(off, R)], kb)

            def build(i, bcarry):
                sl = pl.ds(i * 16, 16)
                ilist[sl] = ilist[sl] + r4e
                return bcarry

            lax.fori_loop(0, R * 4 // 16, build, 0, unroll=8)

            cps = [
                pltpu.async_copy(
                    ybig.at[ilist.at[pl.ds(q * R, R)]],
                    gb.at[pl.ds(q * R, R)], sem)
                for q in range(N_NEI)
            ]
            for cp in cps:
                cp.wait()

            def row(r, rcarry):
                for c in (0, 16):
                    sl = pl.ds(c, 16)
                    v = (sb[r, sl] + gb[4 * r, sl] + gb[4 * r + 1, sl]
                         + gb[4 * r + 2, sl] + gb[4 * r + 3, sl])
                    v = jnp.maximum(v, 0.0)
                    if has_skip:
                        v = jnp.maximum(v + kb[r, sl], 0.0)
                    sb[r, sl] = v
                return rcarry

            lax.fori_loop(0, R, row, 0, unroll=2)
            pltpu.sync_copy(sb, out_hbm.at[pl.ds(off, R)])
            return carry

        nch = NCHUNK + (1 if R_TAIL else 0)
        lax.fori_loop(0, nch, chunk, 0)

    scratch = [
        pltpu.VMEM((N_NEI * R,), jnp.int32),
        pltpu.VMEM((N_NEI * R, MID), jnp.float32),
        pltpu.VMEM((R, MID), jnp.float32),
    ]
    if has_skip:
        scratch.append(pltpu.VMEM((R, MID), jnp.float32))
    scratch.append(pltpu.SemaphoreType.DMA)

    return functools.partial(
        pl.kernel,
        mesh=mesh,
        out_type=jax.ShapeDtypeStruct((E, MID), jnp.float32),
        scratch_types=scratch,
        compiler_params=pltpu.CompilerParams(use_tc_tiling_on_sc=False),
    )(body)


_sc_combine = _make_sc_combine(False)
_sc_combine_skip = _make_sc_combine(True)


SEGP = (E // POOL) // 4


def _pool_body(h_ref, wf_ref, bf_ref, o_ref):
    p = pl.program_id(0)
    part = jnp.zeros((1, CAT), jnp.float32)
    for q in range(4):
        m = jnp.mean(h_ref[q * SEGP:(q + 1) * SEGP, :], axis=0)
        m = (m[0:32] + m[32:64] + m[64:96] + m[96:128]).reshape(1, MID) * 0.25
        part += jnp.dot(m, wf_ref[q * MID:(q + 1) * MID, :],
                        preferred_element_type=jnp.float32)

    @pl.when(p == 0)
    def _():
        o_ref[...] = bf_ref[...]

    o_ref[...] += part


def _pool_fc(h_pk, wf, bf):
    out = pl.pallas_call(
        _pool_body,
        grid=(POOL // 4,),
        in_specs=[
            pl.BlockSpec((4 * SEGP, 128), lambda p: (p, 0)),
            pl.BlockSpec((4 * MID, CAT), lambda p: (p, 0)),
            pl.BlockSpec((1, CAT), lambda p: (0, 0)),
        ],
        out_specs=pl.BlockSpec((1, CAT), lambda p: (0, 0)),
        out_shape=jax.ShapeDtypeStruct((1, CAT), jnp.float32),
    )(h_pk, wf, bf.reshape(1, CAT))
    return out.reshape(CAT)


def _prep(w, b, c):
    eye = jnp.eye(4, dtype=jnp.float32)
    blocks = []
    for k in range(1 + N_NEI):
        wk = w[k * c:(k + 1) * c, :]
        if c < MID:
            wk = jnp.concatenate(
                [wk, jnp.zeros((MID - c, MID), jnp.float32)], axis=0)
        blocks.append(jnp.kron(eye, wk))
    wstack = jnp.concatenate(blocks, axis=0)
    b4 = jnp.tile(b, 4).reshape(1, 128)
    return wstack, b4


def _conv(x_pk, he_flat, w, b, c, skip=None):
    wstack, b4 = _prep(w, b, c)
    s, ybig = _tc_projections(x_pk, wstack, b4)
    if skip is None:
        return _sc_combine(s, ybig, he_flat)
    return _sc_combine_skip(s, ybig, he_flat, skip)


def kernel(x, half_edges, W0, b0, W11, b11, W12, b12, W21, b21, W22, b22, Wf, bf):
    he_flat = half_edges.reshape(N_NEI * E)
    x_pk = jnp.concatenate(
        [x, jnp.zeros((E, MID - IN_C), jnp.float32)], axis=1).reshape(EP, 128)

    h = _conv(x_pk, he_flat, W0, b0, IN_C)
    for (wa, ba, wb, bb) in ((W11, b11, W12, b12), (W21, b21, W22, b22)):
        y = _conv(h.reshape(EP, 128), he_flat, wa, ba, MID)
        h = _conv(y.reshape(EP, 128), he_flat, wb, bb, MID, skip=h)
    return _pool_fc(h.reshape(EP, 128), Wf, bf)


# speedup vs baseline: 2.3971x; 1.1849x over previous
import functools

import jax
import jax.numpy as jnp
from jax import lax
from jax.experimental import pallas as pl
from jax.experimental.pallas import tpu as pltpu
from jax.experimental.pallas import tpu_sc as plsc

E = 800000
N_NEI = 4
IN_C = 16
MID = 32
POOL = 32
CAT = 40

NW = 32
ROWS_W = E // NW
R = 256
NCHUNK = ROWS_W // R
R_TAIL = ROWS_W - NCHUNK * R

BM = 8000
BMP = BM // 4
EP = E // 4


def _proj_body(x_ref, ws_ref, wj_ref, b_ref, s_ref, y_ref):
    x = x_ref[...]
    j = pl.program_id(1)

    @pl.when(j == 0)
    def _():
        s_ref[...] = (
            jnp.dot(x, ws_ref[...], preferred_element_type=jnp.float32)
            + b_ref[...]
        )

    y_ref[...] = jnp.dot(x, wj_ref[...], preferred_element_type=jnp.float32)


def _tc_projections(x_pk, wstack, b4):
    nb = E // BM
    s, ybig = pl.pallas_call(
        _proj_body,
        grid=(nb, N_NEI),
        in_specs=[
            pl.BlockSpec((BMP, 128), lambda i, j: (i, 0)),
            pl.BlockSpec((128, 128), lambda i, j: (0, 0)),
            pl.BlockSpec((128, 128), lambda i, j: (j + 1, 0)),
            pl.BlockSpec((1, 128), lambda i, j: (0, 0)),
        ],
        out_specs=(
            pl.BlockSpec((BMP, 128), lambda i, j: (i, 0)),
            pl.BlockSpec((BMP, 128), lambda i, j: (j * nb + i, 0)),
        ),
        out_shape=(
            jax.ShapeDtypeStruct((EP, 128), jnp.float32),
            jax.ShapeDtypeStruct((N_NEI * EP, 128), jnp.float32),
        ),
    )(x_pk, wstack, wstack, b4)
    return s.reshape(E, MID), ybig.reshape(N_NEI * E, MID)


def _make_sc_combine(has_skip):
    mesh = plsc.VectorSubcoreMesh(core_axis_name="c", subcore_axis_name="s")

    def body(*refs):
        if has_skip:
            (s_hbm, ybig, he_hbm, skip_hbm, out_hbm,
             il0, il1, gb0, gb1, sb0, sb1, kb0, kb1,
             sa0, sa1, sc0, sc1, so0, so1) = refs
            kbs = (kb0, kb1)
        else:
            (s_hbm, ybig, he_hbm, out_hbm,
             il0, il1, gb0, gb1, sb0, sb1,
             sa0, sa1, sc0, sc1, so0, so1) = refs
            kbs = (None, None)
        ils = (il0, il1)
        gbs = (gb0, gb1)
        sbs = (sb0, sb1)
        sas = (sa0, sa1)
        scs = (sc0, sc1)
        sos = (so0, so1)
        wid = lax.axis_index("s") * 2 + lax.axis_index("c")
        base = wid * ROWS_W
        nch = NCHUNK + (1 if R_TAIL else 0)

        iota = lax.iota(jnp.int32, 16)
        r4e = (iota & 3) * E

        def off_of(k):
            return base + jnp.minimum(k * R, ROWS_W - R)

        def fire_in(k, p):
            off = off_of(k)
            pltpu.async_copy(he_hbm.at[pl.ds(off * 4, R * 4)], ils[p], sas[p])
            pltpu.async_copy(s_hbm.at[pl.ds(off, R)], sbs[p], sas[p])
            if has_skip:
                pltpu.async_copy(skip_hbm.at[pl.ds(off, R)], kbs[p], sas[p])

        def wait_in(p):
            pltpu.make_async_copy(he_hbm.at[pl.ds(0, R * 4)], ils[p],
                                  sas[p]).wait()
            pltpu.make_async_copy(s_hbm.at[pl.ds(0, R)], sbs[p],
                                  sas[p]).wait()
            if has_skip:
                pltpu.make_async_copy(skip_hbm.at[pl.ds(0, R)], kbs[p],
                                      sas[p]).wait()

        def build_fire_gather(p):
            il = ils[p]

            def build(i, bcarry):
                sl = pl.ds(i * 16, 16)
                il[sl] = il[sl] + r4e
                return bcarry

            lax.fori_loop(0, R * 4 // 16, build, 0, unroll=8)
            for q in range(N_NEI):
                pltpu.async_copy(ybig.at[il.at[pl.ds(q * R, R)]],
                                 gbs[p].at[pl.ds(q * R, R)], scs[p])

        def wait_gather(p):
            pltpu.make_async_copy(ybig.at[pl.ds(0, N_NEI * R)], gbs[p],
                                  scs[p]).wait()

        def compute_out(k, p):
            sb, gb, kb = sbs[p], gbs[p], kbs[p]

            def row(r, rcarry):
                for c in (0, 16):
                    sl = pl.ds(c, 16)
                    v = (sb[r, sl] + gb[4 * r, sl] + gb[4 * r + 1, sl]
                         + gb[4 * r + 2, sl] + gb[4 * r + 3, sl])
                    v = jnp.maximum(v, 0.0)
                    if has_skip:
                        v = jnp.maximum(v + kb[r, sl], 0.0)
                    sb[r, sl] = v
                return rcarry

            lax.fori_loop(0, R, row, 0, unroll=2)
            pltpu.async_copy(sb, out_hbm.at[pl.ds(off_of(k), R)], sos[p])

        def wait_out(p):
            pltpu.make_async_copy(s_hbm.at[pl.ds(0, R)], sbs[p],
                                  sos[p]).wait()

        fire_in(0, 0)
        fire_in(1, 1)
        wait_in(0)
        build_fire_gather(0)

        def pair(t2, carry):
            for p in (0, 1):
                k = 2 * t2 + p
                nxt = 1 - p

                @pl.when(k + 1 < nch)
                def _():
                    wait_in(nxt)
                    build_fire_gather(nxt)

                wait_gather(p)
                compute_out(k, p)

                @pl.when(k + 2 < nch)
                def _():
                    wait_out(p)
                    fire_in(k + 2, p)
            return carry

        lax.fori_loop(0, nch // 2, pair, 0)
        wait_out(0)
        wait_out(1)

    scratch = [pltpu.VMEM((N_NEI * R,), jnp.int32)] * 2
    scratch += [pltpu.VMEM((N_NEI * R, MID), jnp.float32)] * 2
    scratch += [pltpu.VMEM((R, MID), jnp.float32)] * 2
    if has_skip:
        scratch += [pltpu.VMEM((R, MID), jnp.float32)] * 2
    scratch += [pltpu.SemaphoreType.DMA] * 6

    return functools.partial(
        pl.kernel,
        mesh=mesh,
        out_type=jax.ShapeDtypeStruct((E, MID), jnp.float32),
        scratch_types=scratch,
        compiler_params=pltpu.CompilerParams(use_tc_tiling_on_sc=False),
    )(body)


_sc_combine = _make_sc_combine(False)
_sc_combine_skip = _make_sc_combine(True)


SEGP = (E // POOL) // 4


def _pool_body(h_ref, wf_ref, bf_ref, o_ref):
    p = pl.program_id(0)
    part = jnp.zeros((1, CAT), jnp.float32)
    for q in range(4):
        m = jnp.mean(h_ref[q * SEGP:(q + 1) * SEGP, :], axis=0)
        m = (m[0:32] + m[32:64] + m[64:96] + m[96:128]).reshape(1, MID) * 0.25
        part += jnp.dot(m, wf_ref[q * MID:(q + 1) * MID, :],
                        preferred_element_type=jnp.float32)

    @pl.when(p == 0)
    def _():
        o_ref[...] = bf_ref[...]

    o_ref[...] += part


def _pool_fc(h_pk, wf, bf):
    out = pl.pallas_call(
        _pool_body,
        grid=(POOL // 4,),
        in_specs=[
            pl.BlockSpec((4 * SEGP, 128), lambda p: (p, 0)),
            pl.BlockSpec((4 * MID, CAT), lambda p: (p, 0)),
            pl.BlockSpec((1, CAT), lambda p: (0, 0)),
        ],
        out_specs=pl.BlockSpec((1, CAT), lambda p: (0, 0)),
        out_shape=jax.ShapeDtypeStruct((1, CAT), jnp.float32),
    )(h_pk, wf, bf.reshape(1, CAT))
    return out.reshape(CAT)


def _prep(w, b, c):
    eye = jnp.eye(4, dtype=jnp.float32)
    blocks = []
    for k in range(1 + N_NEI):
        wk = w[k * c:(k + 1) * c, :]
        if c < MID:
            wk = jnp.concatenate(
                [wk, jnp.zeros((MID - c, MID), jnp.float32)], axis=0)
        blocks.append(jnp.kron(eye, wk))
    wstack = jnp.concatenate(blocks, axis=0)
    b4 = jnp.tile(b, 4).reshape(1, 128)
    return wstack, b4


def _conv(x_pk, he_flat, w, b, c, skip=None):
    wstack, b4 = _prep(w, b, c)
    s, ybig = _tc_projections(x_pk, wstack, b4)
    if skip is None:
        return _sc_combine(s, ybig, he_flat)
    return _sc_combine_skip(s, ybig, he_flat, skip)


def kernel(x, half_edges, W0, b0, W11, b11, W12, b12, W21, b21, W22, b22, Wf, bf):
    he_flat = half_edges.reshape(N_NEI * E)
    x_pk = jnp.concatenate(
        [x, jnp.zeros((E, MID - IN_C), jnp.float32)], axis=1).reshape(EP, 128)

    h = _conv(x_pk, he_flat, W0, b0, IN_C)
    for (wa, ba, wb, bb) in ((W11, b11, W12, b12), (W21, b21, W22, b22)):
        y = _conv(h.reshape(EP, 128), he_flat, wa, ba, MID)
        h = _conv(y.reshape(EP, 128), he_flat, wb, bb, MID, skip=h)
    return _pool_fc(h.reshape(EP, 128), Wf, bf)


# speedup vs baseline: 2.4002x; 1.0013x over previous
import functools

import jax
import jax.numpy as jnp
from jax import lax
from jax.experimental import pallas as pl
from jax.experimental.pallas import tpu as pltpu
from jax.experimental.pallas import tpu_sc as plsc

E = 800000
N_NEI = 4
IN_C = 16
MID = 32
POOL = 32
CAT = 40

NW = 32
ROWS_W = E // NW
R = 320
NCH = -(-ROWS_W // R)
NCH += NCH % 2

BM = 8000
BMP = BM // 4
EP = E // 4


def _proj_body(x_ref, ws_ref, wj_ref, b_ref, s_ref, y_ref):
    x = x_ref[...]
    j = pl.program_id(1)

    @pl.when(j == 0)
    def _():
        s_ref[...] = (
            jnp.dot(x, ws_ref[...], preferred_element_type=jnp.float32)
            + b_ref[...]
        )

    y_ref[...] = jnp.dot(x, wj_ref[...], preferred_element_type=jnp.float32)


def _tc_projections(x_pk, wstack, b4):
    nb = E // BM
    s, ybig = pl.pallas_call(
        _proj_body,
        grid=(nb, N_NEI),
        in_specs=[
            pl.BlockSpec((BMP, 128), lambda i, j: (i, 0)),
            pl.BlockSpec((128, 128), lambda i, j: (0, 0)),
            pl.BlockSpec((128, 128), lambda i, j: (j + 1, 0)),
            pl.BlockSpec((1, 128), lambda i, j: (0, 0)),
        ],
        out_specs=(
            pl.BlockSpec((BMP, 128), lambda i, j: (i, 0)),
            pl.BlockSpec((BMP, 128), lambda i, j: (j * nb + i, 0)),
        ),
        out_shape=(
            jax.ShapeDtypeStruct((EP, 128), jnp.float32),
            jax.ShapeDtypeStruct((N_NEI * EP, 128), jnp.float32),
        ),
    )(x_pk, wstack, wstack, b4)
    return s.reshape(E, MID), ybig.reshape(N_NEI * E, MID)


def _make_sc_combine(has_skip):
    mesh = plsc.VectorSubcoreMesh(core_axis_name="c", subcore_axis_name="s")

    def body(*refs):
        if has_skip:
            (s_hbm, ybig, he_hbm, skip_hbm, out_hbm,
             il0, il1, gb0, gb1, sb0, sb1, kb0, kb1,
             sa0, sa1, sc0, sc1, so0, so1) = refs
            kbs = (kb0, kb1)
        else:
            (s_hbm, ybig, he_hbm, out_hbm,
             il0, il1, gb0, gb1, sb0, sb1,
             sa0, sa1, sc0, sc1, so0, so1) = refs
            kbs = (None, None)
        ils = (il0, il1)
        gbs = (gb0, gb1)
        sbs = (sb0, sb1)
        sas = (sa0, sa1)
        scs = (sc0, sc1)
        sos = (so0, so1)
        wid = lax.axis_index("s") * 2 + lax.axis_index("c")
        base = wid * ROWS_W
        nch = NCH

        iota = lax.iota(jnp.int32, 16)
        r4e = (iota & 3) * E

        def off_of(k):
            return base + jnp.minimum(k * R, ROWS_W - R)

        def fire_in(k, p):
            off = off_of(k)
            pltpu.async_copy(he_hbm.at[pl.ds(off * 4, R * 4)], ils[p], sas[p])
            pltpu.async_copy(s_hbm.at[pl.ds(off, R)], sbs[p], sas[p])
            if has_skip:
                pltpu.async_copy(skip_hbm.at[pl.ds(off, R)], kbs[p], sas[p])

        def wait_in(p):
            pltpu.make_async_copy(he_hbm.at[pl.ds(0, R * 4)], ils[p],
                                  sas[p]).wait()
            pltpu.make_async_copy(s_hbm.at[pl.ds(0, R)], sbs[p],
                                  sas[p]).wait()
            if has_skip:
                pltpu.make_async_copy(skip_hbm.at[pl.ds(0, R)], kbs[p],
                                      sas[p]).wait()

        def build_fire_gather(p):
            il = ils[p]

            def build(i, bcarry):
                sl = pl.ds(i * 16, 16)
                il[sl] = il[sl] + r4e
                return bcarry

            lax.fori_loop(0, R * 4 // 16, build, 0, unroll=8)
            for q in range(N_NEI):
                pltpu.async_copy(ybig.at[il.at[pl.ds(q * R, R)]],
                                 gbs[p].at[pl.ds(q * R, R)], scs[p])

        def wait_gather(p):
            pltpu.make_async_copy(ybig.at[pl.ds(0, N_NEI * R)], gbs[p],
                                  scs[p]).wait()

        def compute_out(k, p):
            sb, gb, kb = sbs[p], gbs[p], kbs[p]

            def row(r, rcarry):
                for c in (0, 16):
                    sl = pl.ds(c, 16)
                    v = (sb[r, sl] + gb[4 * r, sl] + gb[4 * r + 1, sl]
                         + gb[4 * r + 2, sl] + gb[4 * r + 3, sl])
                    v = jnp.maximum(v, 0.0)
                    if has_skip:
                        v = jnp.maximum(v + kb[r, sl], 0.0)
                    sb[r, sl] = v
                return rcarry

            lax.fori_loop(0, R, row, 0, unroll=2)
            pltpu.async_copy(sb, out_hbm.at[pl.ds(off_of(k), R)], sos[p])

        def wait_out(p):
            pltpu.make_async_copy(s_hbm.at[pl.ds(0, R)], sbs[p],
                                  sos[p]).wait()

        fire_in(0, 0)
        fire_in(1, 1)
        wait_in(0)
        build_fire_gather(0)

        def pair(t2, carry):
            for p in (0, 1):
                k = 2 * t2 + p
                nxt = 1 - p

                @pl.when(k + 1 < nch)
                def _():
                    wait_in(nxt)
                    build_fire_gather(nxt)

                wait_gather(p)
                compute_out(k, p)

                @pl.when(k + 2 < nch)
                def _():
                    wait_out(p)
                    fire_in(k + 2, p)
            return carry

        lax.fori_loop(0, nch // 2, pair, 0)
        wait_out(0)
        wait_out(1)

    scratch = [pltpu.VMEM((N_NEI * R,), jnp.int32)] * 2
    scratch += [pltpu.VMEM((N_NEI * R, MID), jnp.float32)] * 2
    scratch += [pltpu.VMEM((R, MID), jnp.float32)] * 2
    if has_skip:
        scratch += [pltpu.VMEM((R, MID), jnp.float32)] * 2
    scratch += [pltpu.SemaphoreType.DMA] * 6

    return functools.partial(
        pl.kernel,
        mesh=mesh,
        out_type=jax.ShapeDtypeStruct((E, MID), jnp.float32),
        scratch_types=scratch,
        compiler_params=pltpu.CompilerParams(use_tc_tiling_on_sc=False),
    )(body)


_sc_combine = _make_sc_combine(False)
_sc_combine_skip = _make_sc_combine(True)


SEGP = (E // POOL) // 4


def _pool_body(h_ref, wf_ref, bf_ref, o_ref):
    p = pl.program_id(0)
    part = jnp.zeros((1, CAT), jnp.float32)
    for q in range(4):
        m = jnp.mean(h_ref[q * SEGP:(q + 1) * SEGP, :], axis=0)
        m = (m[0:32] + m[32:64] + m[64:96] + m[96:128]).reshape(1, MID) * 0.25
        part += jnp.dot(m, wf_ref[q * MID:(q + 1) * MID, :],
                        preferred_element_type=jnp.float32)

    @pl.when(p == 0)
    def _():
        o_ref[...] = bf_ref[...]

    o_ref[...] += part


def _pool_fc(h_pk, wf, bf):
    out = pl.pallas_call(
        _pool_body,
        grid=(POOL // 4,),
        in_specs=[
            pl.BlockSpec((4 * SEGP, 128), lambda p: (p, 0)),
            pl.BlockSpec((4 * MID, CAT), lambda p: (p, 0)),
            pl.BlockSpec((1, CAT), lambda p: (0, 0)),
        ],
        out_specs=pl.BlockSpec((1, CAT), lambda p: (0, 0)),
        out_shape=jax.ShapeDtypeStruct((1, CAT), jnp.float32),
    )(h_pk, wf, bf.reshape(1, CAT))
    return out.reshape(CAT)


def _prep(w, b, c):
    eye = jnp.eye(4, dtype=jnp.float32)
    blocks = []
    for k in range(1 + N_NEI):
        wk = w[k * c:(k + 1) * c, :]
        if c < MID:
            wk = jnp.concatenate(
                [wk, jnp.zeros((MID - c, MID), jnp.float32)], axis=0)
        blocks.append(jnp.kron(eye, wk))
    wstack = jnp.concatenate(blocks, axis=0)
    b4 = jnp.tile(b, 4).reshape(1, 128)
    return wstack, b4


def _conv(x_pk, he_flat, w, b, c, skip=None):
    wstack, b4 = _prep(w, b, c)
    s, ybig = _tc_projections(x_pk, wstack, b4)
    if skip is None:
        return _sc_combine(s, ybig, he_flat)
    return _sc_combine_skip(s, ybig, he_flat, skip)


def kernel(x, half_edges, W0, b0, W11, b11, W12, b12, W21, b21, W22, b22, Wf, bf):
    he_flat = half_edges.reshape(N_NEI * E)
    x_pk = jnp.concatenate(
        [x, jnp.zeros((E, MID - IN_C), jnp.float32)], axis=1).reshape(EP, 128)

    h = _conv(x_pk, he_flat, W0, b0, IN_C)
    for (wa, ba, wb, bb) in ((W11, b11, W12, b12), (W21, b21, W22, b22)):
        y = _conv(h.reshape(EP, 128), he_flat, wa, ba, MID)
        h = _conv(y.reshape(EP, 128), he_flat, wb, bb, MID, skip=h)
    return _pool_fc(h.reshape(EP, 128), Wf, bf)
